# baseline (speedup 1.0000x reference)
import functools

import jax
import jax.numpy as jnp
from jax import lax
from jax.experimental import pallas as pl
from jax.experimental.pallas import tpu as pltpu
from jax.experimental.pallas import tpu_sc as plsc

_BATCH = 16384
_D = 32
_L = 16
_NC, _NS = 2, 16
_NW = _NC * _NS
_ROWS_PER_W = _BATCH // _NW
_CHUNK = 128
_NCHUNK = _ROWS_PER_W // _CHUNK
_LAMBDA = 1e-05


def _sc_body(h_hbm, r_hbm, pos_hbm, neg_hbm,
             ent_re_hbm, ent_im_hbm, rel_re_hbm, rel_im_hbm,
             ddiff_hbm, sq_hbm,
             hidx, ridx, pidx, nidx,
             hre, him, trep, timp, tren, timn, rre, rim,
             dbuf, sqbuf, sem):
    wid = lax.axis_index("s") * _NC + lax.axis_index("c")
    acc = jnp.zeros((_L,), jnp.float32)
    for c in range(_NCHUNK):
        base = wid * _ROWS_PER_W + c * _CHUNK
        pltpu.sync_copy(h_hbm.at[pl.ds(base, _CHUNK)], hidx)
        pltpu.sync_copy(r_hbm.at[pl.ds(base, _CHUNK)], ridx)
        pltpu.sync_copy(pos_hbm.at[pl.ds(base, _CHUNK)], pidx)
        pltpu.sync_copy(neg_hbm.at[pl.ds(base, _CHUNK)], nidx)
        copies = [
            pltpu.async_copy(ent_re_hbm.at[hidx], hre, sem),
            pltpu.async_copy(ent_im_hbm.at[hidx], him, sem),
            pltpu.async_copy(ent_re_hbm.at[pidx], trep, sem),
            pltpu.async_copy(ent_im_hbm.at[pidx], timp, sem),
            pltpu.async_copy(ent_re_hbm.at[nidx], tren, sem),
            pltpu.async_copy(ent_im_hbm.at[nidx], timn, sem),
            pltpu.async_copy(rel_re_hbm.at[ridx], rre, sem),
            pltpu.async_copy(rel_im_hbm.at[ridx], rim, sem),
        ]
        for cp in copies:
            cp.wait()

        def row_body(i, a):
            h0 = hre[i, pl.ds(0, _L)]
            h1 = hre[i, pl.ds(_L, _L)]
            m0 = him[i, pl.ds(0, _L)]
            m1 = him[i, pl.ds(_L, _L)]
            r0 = rre[i, pl.ds(0, _L)]
            r1 = rre[i, pl.ds(_L, _L)]
            s0 = rim[i, pl.ds(0, _L)]
            s1 = rim[i, pl.ds(_L, _L)]
            p0 = trep[i, pl.ds(0, _L)]
            p1 = trep[i, pl.ds(_L, _L)]
            q0 = timp[i, pl.ds(0, _L)]
            q1 = timp[i, pl.ds(_L, _L)]
            u0 = tren[i, pl.ds(0, _L)]
            u1 = tren[i, pl.ds(_L, _L)]
            v0 = timn[i, pl.ds(0, _L)]
            v1 = timn[i, pl.ds(_L, _L)]
            a0 = h0 * r0 - m0 * s0
            a1 = h1 * r1 - m1 * s1
            b0 = m0 * r0 + h0 * s0
            b1 = m1 * r1 + h1 * s1
            d0 = a0 * (u0 - p0) + b0 * (v0 - q0)
            d1 = a1 * (u1 - p1) + b1 * (v1 - q1)
            dbuf[i, :] = d0 + d1
            sq = (h0 * h0 + h1 * h1 + m0 * m0 + m1 * m1
                  + p0 * p0 + p1 * p1 + q0 * q0 + q1 * q1
                  + u0 * u0 + u1 * u1 + v0 * v0 + v1 * v1
                  + r0 * r0 + r1 * r1 + s0 * s0 + s1 * s1)
            return a + sq

        acc = lax.fori_loop(0, _CHUNK, row_body, acc)
        pltpu.sync_copy(dbuf, ddiff_hbm.at[pl.ds(base, _CHUNK)])
    sqbuf[...] = acc
    pltpu.sync_copy(sqbuf, sq_hbm.at[wid])


_sc_kernel = functools.partial(
    pl.kernel,
    out_type=[
        jax.ShapeDtypeStruct((_BATCH, _L), jnp.float32),
        jax.ShapeDtypeStruct((_NW, _L), jnp.float32),
    ],
    mesh=plsc.VectorSubcoreMesh(core_axis_name="c", subcore_axis_name="s"),
    compiler_params=pltpu.CompilerParams(use_tc_tiling_on_sc=False),
    scratch_types=[
        pltpu.VMEM((_CHUNK,), jnp.int32),
        pltpu.VMEM((_CHUNK,), jnp.int32),
        pltpu.VMEM((_CHUNK,), jnp.int32),
        pltpu.VMEM((_CHUNK,), jnp.int32),
    ] + [pltpu.VMEM((_CHUNK, _D), jnp.float32) for _ in range(8)] + [
        pltpu.VMEM((_CHUNK, _L), jnp.float32),
        pltpu.VMEM((_L,), jnp.float32),
        pltpu.SemaphoreType.DMA,
    ],
)(_sc_body)


def _tc_body(dd_ref, sq_ref, out_ref):
    x = dd_ref[...]
    rs = jnp.sum(x, axis=1, keepdims=True)
    t = jnp.maximum(-rs, 0.0) + jnp.log1p(jnp.exp(-jnp.abs(rs)))
    l2 = jnp.sum(sq_ref[...])
    out_ref[0, 0] = jnp.sum(t) * (1.0 / _BATCH) + l2 * (_LAMBDA / (2.0 * _BATCH))


def kernel(h, r, pos_t, neg_t, ent_re, ent_im, rel_re, rel_im):
    ddiff, sq = _sc_kernel(h, r, pos_t, neg_t, ent_re, ent_im, rel_re, rel_im)
    loss = pl.pallas_call(
        _tc_body,
        out_shape=jax.ShapeDtypeStruct((1, 1), jnp.float32),
        out_specs=pl.BlockSpec(memory_space=pltpu.SMEM),
    )(ddiff, sq)
    return loss[0, 0]
